# Initial kernel scaffold; baseline (speedup 1.0000x reference)
import jax
import jax.numpy as jnp
from jax.experimental import pallas as pl


def kernel(x, vals, ohe):
    raise NotImplementedError("write your pallas kernel here")


# speedup vs baseline: 1.6814x; 1.6814x over previous
import functools

import jax
import jax.numpy as jnp
from jax import lax
from jax.experimental import pallas as pl
from jax.experimental.pallas import tpu as pltpu
from jax.experimental.pallas import tpu_sc as plsc

BATCH = 4096
FIELDS = 26
D = 1000
B = BATCH * FIELDS

NC, NS = 2, 16
NW = NC * NS
B_PER_W = B // NW
CHUNK = 64
N_STEPS = B_PER_W // CHUNK

_sc_mesh = plsc.VectorSubcoreMesh(core_axis_name="c", subcore_axis_name="s")


@functools.partial(
    pl.kernel,
    mesh=_sc_mesh,
    compiler_params=pltpu.CompilerParams(use_tc_tiling_on_sc=False),
    out_type=jax.ShapeDtypeStruct((B, D), jnp.float32),
    scratch_types=[
        pltpu.VMEM((CHUNK,), jnp.int32),
        pltpu.VMEM((CHUNK, D), jnp.float32),
        pltpu.SemaphoreType.DMA,
    ],
)
def _gather_sc(idx_hbm, table_hbm, out_hbm, idx_v, rows_v, sem):
    wid = lax.axis_index("s") * NC + lax.axis_index("c")
    base0 = wid * B_PER_W

    def body(i, carry):
        base = base0 + i * CHUNK
        pltpu.sync_copy(idx_hbm.at[pl.ds(base, CHUNK)], idx_v)
        pltpu.async_copy(table_hbm.at[idx_v], rows_v, sem).wait()
        pltpu.sync_copy(rows_v, out_hbm.at[pl.ds(base, CHUNK)])
        return carry

    lax.fori_loop(0, N_STEPS, body, 0)


def _bool_body(x_ref, o_ref):
    idx = x_ref[...] - 1
    lane = lax.broadcasted_iota(jnp.int32, o_ref.shape, 2)
    o_ref[...] = (lane == idx).astype(jnp.float32)


_BB = 128


def _onehot_tc(x3):
    return pl.pallas_call(
        _bool_body,
        grid=(BATCH // _BB,),
        in_specs=[pl.BlockSpec((_BB, FIELDS, 1), lambda i: (i, 0, 0))],
        out_specs=pl.BlockSpec((_BB, FIELDS, D), lambda i: (i, 0, 0)),
        out_shape=jax.ShapeDtypeStruct((BATCH, FIELDS, D), jnp.float32),
    )(x3)


def kernel(x, vals, ohe):
    xi = x.astype(jnp.int32)
    embed = _gather_sc(xi.reshape(B), vals)
    embed_bool = _onehot_tc(xi[:, :, None])
    return (embed.reshape(BATCH, FIELDS * D),
            embed_bool.reshape(BATCH, FIELDS * D))


# speedup vs baseline: 1.8120x; 1.0776x over previous
import functools

import jax
import jax.numpy as jnp
from jax import lax
from jax.experimental import pallas as pl
from jax.experimental.pallas import tpu as pltpu
from jax.experimental.pallas import tpu_sc as plsc

BATCH = 4096
FIELDS = 26
D = 1000
B = BATCH * FIELDS

NC, NS = 2, 16
NW = NC * NS
B_PER_W = B // NW
CHUNK = 64
N_STEPS = B_PER_W // CHUNK

_sc_mesh = plsc.VectorSubcoreMesh(core_axis_name="c", subcore_axis_name="s")


@functools.partial(
    pl.kernel,
    mesh=_sc_mesh,
    compiler_params=pltpu.CompilerParams(use_tc_tiling_on_sc=False),
    out_type=jax.ShapeDtypeStruct((B, D), jnp.float32),
    scratch_types=[
        pltpu.VMEM((CHUNK,), jnp.int32),
        pltpu.VMEM((CHUNK, D), jnp.float32),
        pltpu.SemaphoreType.DMA,
    ],
)
def _gather_sc(idx_hbm, table_hbm, out_hbm, idx_v, rows_v, sem):
    wid = lax.axis_index("s") * NC + lax.axis_index("c")
    base0 = wid * B_PER_W

    def body(i, carry):
        base = base0 + i * CHUNK
        pltpu.sync_copy(idx_hbm.at[pl.ds(base, CHUNK)], idx_v)
        pltpu.async_copy(table_hbm.at[idx_v], rows_v, sem).wait()
        pltpu.sync_copy(rows_v, out_hbm.at[pl.ds(base, CHUNK)])
        return carry

    lax.fori_loop(0, N_STEPS, body, 0)


def _bool_body(x_ref, o_ref):
    xv = x_ref[...]
    lane = lax.broadcasted_iota(jnp.int32, (o_ref.shape[0], D), 1)
    for f in range(FIELDS):
        idx = xv[:, f:f + 1] - 1
        o_ref[:, f * D:(f + 1) * D] = (lane == idx).astype(jnp.float32)


_BB = 128


def _onehot_tc(xi):
    return pl.pallas_call(
        _bool_body,
        grid=(BATCH // _BB,),
        in_specs=[pl.BlockSpec((_BB, FIELDS), lambda i: (i, 0))],
        out_specs=pl.BlockSpec((_BB, FIELDS * D), lambda i: (i, 0)),
        out_shape=jax.ShapeDtypeStruct((BATCH, FIELDS * D), jnp.float32),
    )(xi)


def kernel(x, vals, ohe):
    xi = x.astype(jnp.int32)
    embed = _gather_sc(xi.reshape(B), vals)
    embed_bool = _onehot_tc(xi)
    return embed.reshape(BATCH, FIELDS * D), embed_bool
